# baseline (speedup 1.0000x reference)
import functools

import jax
import jax.numpy as jnp
from jax.experimental import pallas as pl
from jax.experimental.pallas import tpu as pltpu

B, N, D = 32, 1024, 768
S = 8
H = 1536
ITERS = 3
EPS = 1e-8


def _ln(x, g, b):
    m = jnp.mean(x, axis=-1, keepdims=True)
    v = jnp.mean((x - m) ** 2, axis=-1, keepdims=True)
    return (x - m) * jax.lax.rsqrt(v + 1e-5) * g + b


def _sa_kernel(x_ref, noise_ref, mu_ref, sigma_ref,
               WqT_ref, bq_ref, WkT_ref, bk_ref, WvT_ref, bv_ref,
               WihT_ref, WhhT_ref, bih_ref, bhh_ref,
               W1T_ref, b1_ref, W2T_ref, b2_ref,
               g_in_ref, b_in_ref, g_s_ref, b_s_ref, g_ff_ref, b_ff_ref,
               out_ref):
    scale = D ** -0.5
    bf = jnp.bfloat16
    x = x_ref[0]
    xh = _ln(x, g_in_ref[...], b_in_ref[...]).astype(bf)
    k = jnp.dot(xh, WkT_ref[...].astype(bf), preferred_element_type=jnp.float32) + bk_ref[...]
    v = jnp.dot(xh, WvT_ref[...].astype(bf), preferred_element_type=jnp.float32) + bv_ref[...]
    kb = k.astype(bf)
    vb = v.astype(bf)

    slots = mu_ref[0] + sigma_ref[0] * noise_ref[0]

    for _ in range(ITERS):
        slots_prev = slots
        slots_n = _ln(slots, g_s_ref[...], b_s_ref[...]).astype(bf)
        q = jnp.dot(slots_n, WqT_ref[...].astype(bf), preferred_element_type=jnp.float32) + bq_ref[...]
        dots = jax.lax.dot_general(
            q.astype(bf), kb, (((1,), (1,)), ((), ())),
            preferred_element_type=jnp.float32) * scale
        dmax = jnp.max(dots, axis=0, keepdims=True)
        e = jnp.exp(dots - dmax)
        attn = e / jnp.sum(e, axis=0, keepdims=True) + EPS
        attn = attn / jnp.sum(attn, axis=1, keepdims=True)
        updates = jnp.dot(attn.astype(bf), vb, preferred_element_type=jnp.float32)

        gi = jnp.dot(updates.astype(bf), WihT_ref[...].astype(bf), preferred_element_type=jnp.float32) + bih_ref[...]
        gh = jnp.dot(slots_prev.astype(bf), WhhT_ref[...].astype(bf), preferred_element_type=jnp.float32) + bhh_ref[...]
        r = jax.nn.sigmoid(gi[:, :D] + gh[:, :D])
        z = jax.nn.sigmoid(gi[:, D:2 * D] + gh[:, D:2 * D])
        n_ = jnp.tanh(gi[:, 2 * D:] + r * gh[:, 2 * D:])
        slots = (1.0 - z) * n_ + z * slots_prev

        ffx = _ln(slots, g_ff_ref[...], b_ff_ref[...]).astype(bf)
        ff = jnp.dot(jax.nn.relu(
            jnp.dot(ffx, W1T_ref[...].astype(bf), preferred_element_type=jnp.float32) + b1_ref[...]).astype(bf),
            W2T_ref[...].astype(bf), preferred_element_type=jnp.float32) + b2_ref[...]
        slots = slots + ff

    out_ref[0] = slots


@jax.jit
def kernel(x, slots_noise, mu, logsigma, Wq, bq, Wk, bk, Wv, bv,
           W_ih, W_hh, b_ih, b_hh, W1, b1, W2, b2,
           g_in, b_in, g_slots, b_slots, g_ff, b_ff):
    row = lambda a: a.reshape(1, -1)
    full = lambda s: pl.BlockSpec(s, lambda b: (0,) * len(s))
    args = (
        x, slots_noise, mu, jnp.exp(logsigma),
        Wq.T, row(bq), Wk.T, row(bk), Wv.T, row(bv),
        W_ih.T, W_hh.T, row(b_ih), row(b_hh),
        W1.T, row(b1), W2.T, row(b2),
        row(g_in), row(b_in), row(g_slots), row(b_slots), row(g_ff), row(b_ff),
    )
    in_specs = [
        pl.BlockSpec((1, N, D), lambda b: (b, 0, 0)),
        pl.BlockSpec((1, S, D), lambda b: (b, 0, 0)),
        full((1, 1, D)), full((1, 1, D)),
        full((D, D)), full((1, D)), full((D, D)), full((1, D)),
        full((D, D)), full((1, D)),
        full((D, 3 * D)), full((D, 3 * D)), full((1, 3 * D)), full((1, 3 * D)),
        full((D, H)), full((1, H)), full((H, D)), full((1, D)),
        full((1, D)), full((1, D)), full((1, D)), full((1, D)),
        full((1, D)), full((1, D)),
    ]
    out = pl.pallas_call(
        _sa_kernel,
        grid=(B,),
        in_specs=in_specs,
        out_specs=pl.BlockSpec((1, S, D), lambda b: (b, 0, 0)),
        out_shape=jax.ShapeDtypeStruct((B, S, D), jnp.float32),
    )(*args)
    return out


# speedup vs baseline: 1.5279x; 1.5279x over previous
import jax
import jax.numpy as jnp
from jax.experimental import pallas as pl
from jax.experimental.pallas import tpu as pltpu

B, N, D = 32, 1024, 768
S = 8
H = 1536
ITERS = 3
EPS = 1e-8
BS = B * S


def _ln(x, g, b):
    m = jnp.mean(x, axis=-1, keepdims=True)
    v = jnp.mean((x - m) ** 2, axis=-1, keepdims=True)
    return (x - m) * jax.lax.rsqrt(v + 1e-5) * g + b


def _proj_kernel(x_ref, WkT_ref, bk_ref, WvT_ref, bv_ref, g_in_ref, b_in_ref,
                 k_ref, v_ref):
    bf = jnp.bfloat16
    xh = _ln(x_ref[0], g_in_ref[...], b_in_ref[...]).astype(bf)
    k_ref[0] = (jnp.dot(xh, WkT_ref[...], preferred_element_type=jnp.float32)
                + bk_ref[...]).astype(bf)
    v_ref[0] = (jnp.dot(xh, WvT_ref[...], preferred_element_type=jnp.float32)
                + bv_ref[...]).astype(bf)


def _iter_kernel(k_ref, v_ref, noise_ref, mu_ref, sigma_ref,
                 WqT_ref, bq_ref, WihT_ref, WhhT_ref, bih_ref, bhh_ref,
                 W1T_ref, b1_ref, W2T_ref, b2_ref,
                 g_s_ref, b_s_ref, g_ff_ref, b_ff_ref,
                 out_ref, slots_sc, upd_sc, q_sc):
    bf = jnp.bfloat16
    i = pl.program_id(0)
    b = pl.program_id(1)
    scale = D ** -0.5

    @pl.when(jnp.logical_and(i == 0, b == 0))
    def _init():
        slots0 = (mu_ref[0] + sigma_ref[0] * noise_ref[...].reshape(BS, D))
        slots_sc[...] = slots0
        q_sc[...] = jnp.dot(
            _ln(slots0, g_s_ref[...], b_s_ref[...]).astype(bf),
            WqT_ref[...], preferred_element_type=jnp.float32) + bq_ref[...]

    q_b = q_sc[pl.ds(b * S, S), :].astype(bf)
    dots = jax.lax.dot_general(
        q_b, k_ref[0], (((1,), (1,)), ((), ())),
        preferred_element_type=jnp.float32) * scale
    dmax = jnp.max(dots, axis=0, keepdims=True)
    e = jnp.exp(dots - dmax)
    attn = e / jnp.sum(e, axis=0, keepdims=True) + EPS
    attn = attn / jnp.sum(attn, axis=1, keepdims=True)
    upd_sc[pl.ds(b * S, S), :] = jnp.dot(
        attn.astype(bf), v_ref[0], preferred_element_type=jnp.float32)

    @pl.when(b == B - 1)
    def _global():
        slots_prev = slots_sc[...]
        gi = jnp.dot(upd_sc[...].astype(bf), WihT_ref[...],
                     preferred_element_type=jnp.float32) + bih_ref[...]
        gh = jnp.dot(slots_prev.astype(bf), WhhT_ref[...],
                     preferred_element_type=jnp.float32) + bhh_ref[...]
        r = jax.nn.sigmoid(gi[:, :D] + gh[:, :D])
        z = jax.nn.sigmoid(gi[:, D:2 * D] + gh[:, D:2 * D])
        n_ = jnp.tanh(gi[:, 2 * D:] + r * gh[:, 2 * D:])
        slots = (1.0 - z) * n_ + z * slots_prev

        ffx = _ln(slots, g_ff_ref[...], b_ff_ref[...]).astype(bf)
        h1 = jax.nn.relu(jnp.dot(ffx, W1T_ref[...],
                                 preferred_element_type=jnp.float32)
                         + b1_ref[...]).astype(bf)
        slots = slots + jnp.dot(h1, W2T_ref[...],
                                preferred_element_type=jnp.float32) + b2_ref[...]
        slots_sc[...] = slots

        @pl.when(i < ITERS - 1)
        def _next_q():
            q_sc[...] = jnp.dot(
                _ln(slots, g_s_ref[...], b_s_ref[...]).astype(bf),
                WqT_ref[...], preferred_element_type=jnp.float32) + bq_ref[...]

        @pl.when(i == ITERS - 1)
        def _emit():
            out_ref[...] = slots.reshape(B, S, D)


@jax.jit
def kernel(x, slots_noise, mu, logsigma, Wq, bq, Wk, bk, Wv, bv,
           W_ih, W_hh, b_ih, b_hh, W1, b1, W2, b2,
           g_in, b_in, g_slots, b_slots, g_ff, b_ff):
    bf = jnp.bfloat16
    row = lambda a: a.reshape(1, -1)
    full = lambda s, n: pl.BlockSpec(s, lambda *_: (0,) * n)

    k, v = pl.pallas_call(
        _proj_kernel,
        grid=(B,),
        in_specs=[
            pl.BlockSpec((1, N, D), lambda b: (b, 0, 0)),
            full((D, D), 2), full((1, D), 2),
            full((D, D), 2), full((1, D), 2),
            full((1, D), 2), full((1, D), 2),
        ],
        out_specs=[pl.BlockSpec((1, N, D), lambda b: (b, 0, 0))] * 2,
        out_shape=[jax.ShapeDtypeStruct((B, N, D), bf)] * 2,
    )(x, Wk.T.astype(bf), row(bk), Wv.T.astype(bf), row(bv),
      row(g_in), row(b_in))

    out = pl.pallas_call(
        _iter_kernel,
        grid=(ITERS, B),
        in_specs=[
            pl.BlockSpec((1, N, D), lambda i, b: (b, 0, 0)),
            pl.BlockSpec((1, N, D), lambda i, b: (b, 0, 0)),
            full((B, S, D), 3),
            full((1, 1, D), 3), full((1, 1, D), 3),
            full((D, D), 2), full((1, D), 2),
            full((D, 3 * D), 2), full((D, 3 * D), 2),
            full((1, 3 * D), 2), full((1, 3 * D), 2),
            full((D, H), 2), full((1, H), 2),
            full((H, D), 2), full((1, D), 2),
            full((1, D), 2), full((1, D), 2),
            full((1, D), 2), full((1, D), 2),
        ],
        out_specs=full((B, S, D), 3),
        out_shape=jax.ShapeDtypeStruct((B, S, D), jnp.float32),
        scratch_shapes=[
            pltpu.VMEM((BS, D), jnp.float32),
            pltpu.VMEM((BS, D), jnp.float32),
            pltpu.VMEM((BS, D), jnp.float32),
        ],
    )(k, v, slots_noise, mu, jnp.exp(logsigma),
      Wq.T.astype(bf), row(bq),
      W_ih.T.astype(bf), W_hh.T.astype(bf), row(b_ih), row(b_hh),
      W1.T.astype(bf), row(b1), W2.T.astype(bf), row(b2),
      row(g_slots), row(b_slots), row(g_ff), row(b_ff))
    return out


# speedup vs baseline: 1.5344x; 1.0043x over previous
import jax
import jax.numpy as jnp
from jax.experimental import pallas as pl
from jax.experimental.pallas import tpu as pltpu

B, N, D = 32, 1024, 768
S = 8
H = 1536
ITERS = 3
EPS = 1e-8
G = 4
NH = N // 2


def _ln(x, g, b):
    m = jnp.mean(x, axis=-1, keepdims=True)
    v = jnp.mean((x - m) ** 2, axis=-1, keepdims=True)
    return (x - m) * jax.lax.rsqrt(v + 1e-5) * g + b


def _sa_kernel(x_ref, noise_ref, mu_ref, sigma_ref,
               WkT_ref, bk_ref, WvT_ref, bv_ref,
               WqT_ref, bq_ref, WihT_ref, WhhT_ref, bih_ref, bhh_ref,
               W1T_ref, b1_ref, W2T_ref, b2_ref,
               g_in_ref, b_in_ref, g_s_ref, b_s_ref, g_ff_ref, b_ff_ref,
               out_ref, k_sc, v_sc):
    bf = jnp.bfloat16
    h = pl.program_id(1)
    scale = D ** -0.5

    xh = _ln(x_ref[...].reshape(G * NH, D),
             g_in_ref[...], b_in_ref[...]).astype(bf)
    kh = (jnp.dot(xh, WkT_ref[...], preferred_element_type=jnp.float32)
          + bk_ref[...]).astype(bf)
    vh = (jnp.dot(xh, WvT_ref[...], preferred_element_type=jnp.float32)
          + bv_ref[...]).astype(bf)
    k_sc[:, pl.ds(h * NH, NH), :] = kh.reshape(G, NH, D)
    v_sc[:, pl.ds(h * NH, NH), :] = vh.reshape(G, NH, D)

    @pl.when(h == 1)
    def _iterate():
        slots = (mu_ref[0] + sigma_ref[0]
                 * noise_ref[...].reshape(G * S, D))
        for _ in range(ITERS):
            slots_prev = slots
            q_all = (jnp.dot(_ln(slots, g_s_ref[...], b_s_ref[...]).astype(bf),
                             WqT_ref[...], preferred_element_type=jnp.float32)
                     + bq_ref[...]).astype(bf)
            upds = []
            for gi_ in range(G):
                dots = jax.lax.dot_general(
                    q_all[gi_ * S:(gi_ + 1) * S, :], k_sc[gi_],
                    (((1,), (1,)), ((), ())),
                    preferred_element_type=jnp.float32) * scale
                dmax = jnp.max(dots, axis=0, keepdims=True)
                e = jnp.exp(dots - dmax)
                attn = e / jnp.sum(e, axis=0, keepdims=True) + EPS
                attn = attn / jnp.sum(attn, axis=1, keepdims=True)
                upds.append(jnp.dot(attn.astype(bf), v_sc[gi_],
                                    preferred_element_type=jnp.float32))
            updates = jnp.concatenate(upds, axis=0)

            gi = jnp.dot(updates.astype(bf), WihT_ref[...],
                         preferred_element_type=jnp.float32) + bih_ref[...]
            gh = jnp.dot(slots_prev.astype(bf), WhhT_ref[...],
                         preferred_element_type=jnp.float32) + bhh_ref[...]
            r = jax.nn.sigmoid(gi[:, :D] + gh[:, :D])
            z = jax.nn.sigmoid(gi[:, D:2 * D] + gh[:, D:2 * D])
            n_ = jnp.tanh(gi[:, 2 * D:] + r * gh[:, 2 * D:])
            slots = (1.0 - z) * n_ + z * slots_prev

            ffx = _ln(slots, g_ff_ref[...], b_ff_ref[...]).astype(bf)
            h1 = jax.nn.relu(jnp.dot(ffx, W1T_ref[...],
                                     preferred_element_type=jnp.float32)
                             + b1_ref[...]).astype(bf)
            slots = slots + jnp.dot(h1, W2T_ref[...],
                                    preferred_element_type=jnp.float32) \
                + b2_ref[...]

        out_ref[...] = slots.reshape(G, S, D)


@jax.jit
def kernel(x, slots_noise, mu, logsigma, Wq, bq, Wk, bk, Wv, bv,
           W_ih, W_hh, b_ih, b_hh, W1, b1, W2, b2,
           g_in, b_in, g_slots, b_slots, g_ff, b_ff):
    bf = jnp.bfloat16
    row = lambda a: a.reshape(1, -1)
    full = lambda s, n: pl.BlockSpec(s, lambda *_: (0,) * n)

    out = pl.pallas_call(
        _sa_kernel,
        grid=(B // G, 2),
        in_specs=[
            pl.BlockSpec((G, NH, D), lambda g, h: (g, h, 0)),
            pl.BlockSpec((G, S, D), lambda g, h: (g, 0, 0)),
            full((1, 1, D), 3), full((1, 1, D), 3),
            full((D, D), 2), full((1, D), 2),
            full((D, D), 2), full((1, D), 2),
            full((D, D), 2), full((1, D), 2),
            full((D, 3 * D), 2), full((D, 3 * D), 2),
            full((1, 3 * D), 2), full((1, 3 * D), 2),
            full((D, H), 2), full((1, H), 2),
            full((H, D), 2), full((1, D), 2),
            full((1, D), 2), full((1, D), 2),
            full((1, D), 2), full((1, D), 2),
            full((1, D), 2), full((1, D), 2),
        ],
        out_specs=pl.BlockSpec((G, S, D), lambda g, h: (g, 0, 0)),
        out_shape=jax.ShapeDtypeStruct((B, S, D), jnp.float32),
        scratch_shapes=[
            pltpu.VMEM((G, N, D), bf),
            pltpu.VMEM((G, N, D), bf),
        ],
    )(x, slots_noise, mu, jnp.exp(logsigma),
      Wk.T.astype(bf), row(bk), Wv.T.astype(bf), row(bv),
      Wq.T.astype(bf), row(bq),
      W_ih.T.astype(bf), W_hh.T.astype(bf), row(b_ih), row(b_hh),
      W1.T.astype(bf), row(b1), W2.T.astype(bf), row(b2),
      row(g_in), row(b_in), row(g_slots), row(b_slots), row(g_ff), row(b_ff))
    return out


# speedup vs baseline: 1.7581x; 1.1458x over previous
import jax
import jax.numpy as jnp
from jax.experimental import pallas as pl
from jax.experimental.pallas import tpu as pltpu

B, N, D = 32, 1024, 768
S = 8
H = 1536
ITERS = 3
EPS = 1e-8
G = 4
NG = B // G
NH = N // 2
GS = G * S
BS = B * S
SCALE = D ** -0.5


def _ln(x, g, b):
    m = jnp.mean(x, axis=-1, keepdims=True)
    v = jnp.mean((x - m) ** 2, axis=-1, keepdims=True)
    return (x - m) * jax.lax.rsqrt(v + 1e-5) * g + b


def _attend(q_b, k_b, v_b):
    dots = jax.lax.dot_general(
        q_b, k_b, (((1,), (1,)), ((), ())),
        preferred_element_type=jnp.float32) * SCALE
    dmax = jnp.max(dots, axis=0, keepdims=True)
    e = jnp.exp(dots - dmax)
    attn = e / jnp.sum(e, axis=0, keepdims=True) + EPS
    attn = attn / jnp.sum(attn, axis=1, keepdims=True)
    return jnp.dot(attn.astype(jnp.bfloat16), v_b,
                   preferred_element_type=jnp.float32)


def _proj_kernel(x_ref, noise_ref, mu_ref, sigma_ref,
                 WkT_ref, bk_ref, WvT_ref, bv_ref, WqT_ref, bq_ref,
                 g_in_ref, b_in_ref, g_s_ref, b_s_ref,
                 k_ref, v_ref, upd0_ref):
    bf = jnp.bfloat16
    h = pl.program_id(1)
    xh = _ln(x_ref[...].reshape(G * NH, D),
             g_in_ref[...], b_in_ref[...]).astype(bf)
    kh = (jnp.dot(xh, WkT_ref[...], preferred_element_type=jnp.float32)
          + bk_ref[...]).astype(bf)
    vh = (jnp.dot(xh, WvT_ref[...], preferred_element_type=jnp.float32)
          + bv_ref[...]).astype(bf)
    k_ref[:, pl.ds(h * NH, NH), :] = kh.reshape(G, NH, D)
    v_ref[:, pl.ds(h * NH, NH), :] = vh.reshape(G, NH, D)

    @pl.when(h == 1)
    def _attn0():
        slots0 = mu_ref[0] + sigma_ref[0] * noise_ref[...].reshape(GS, D)
        q0 = (jnp.dot(_ln(slots0, g_s_ref[...], b_s_ref[...]).astype(bf),
                      WqT_ref[...], preferred_element_type=jnp.float32)
              + bq_ref[...]).astype(bf)
        for gi_ in range(G):
            upd0_ref[gi_] = _attend(q0[gi_ * S:(gi_ + 1) * S, :],
                                    k_ref[gi_], v_ref[gi_])


def _gru_ff(upd, slots_prev, WihT_ref, WhhT_ref, bih_ref, bhh_ref,
            W1T_ref, b1_ref, W2T_ref, b2_ref, g_ff_ref, b_ff_ref):
    bf = jnp.bfloat16
    gi = jnp.dot(upd.astype(bf), WihT_ref[...],
                 preferred_element_type=jnp.float32) + bih_ref[...]
    gh = jnp.dot(slots_prev.astype(bf), WhhT_ref[...],
                 preferred_element_type=jnp.float32) + bhh_ref[...]
    r = jax.nn.sigmoid(gi[:, :D] + gh[:, :D])
    z = jax.nn.sigmoid(gi[:, D:2 * D] + gh[:, D:2 * D])
    n_ = jnp.tanh(gi[:, 2 * D:] + r * gh[:, 2 * D:])
    slots = (1.0 - z) * n_ + z * slots_prev
    ffx = _ln(slots, g_ff_ref[...], b_ff_ref[...]).astype(bf)
    h1 = jax.nn.relu(jnp.dot(ffx, W1T_ref[...],
                             preferred_element_type=jnp.float32)
                     + b1_ref[...]).astype(bf)
    return slots + jnp.dot(h1, W2T_ref[...],
                           preferred_element_type=jnp.float32) + b2_ref[...]


def _iter_kernel(k_ref, v_ref, upd0_ref, noise_ref, mu_ref, sigma_ref,
                 WqT_ref, bq_ref, WihT_ref, WhhT_ref, bih_ref, bhh_ref,
                 W1T_ref, b1_ref, W2T_ref, b2_ref,
                 g_s_ref, b_s_ref, g_ff_ref, b_ff_ref,
                 out_ref, slots_sc, upd_sc, q_sc):
    bf = jnp.bfloat16
    j = pl.program_id(0)
    g = pl.program_id(1)
    gru_args = (WihT_ref, WhhT_ref, bih_ref, bhh_ref,
                W1T_ref, b1_ref, W2T_ref, b2_ref, g_ff_ref, b_ff_ref)

    @pl.when(jnp.logical_and(j == 0, g == 0))
    def _init():
        slots0 = mu_ref[0] + sigma_ref[0] * noise_ref[...].reshape(BS, D)
        slots = _gru_ff(upd0_ref[...].reshape(BS, D), slots0, *gru_args)
        slots_sc[...] = slots
        q_sc[...] = (jnp.dot(_ln(slots, g_s_ref[...], b_s_ref[...]).astype(bf),
                             WqT_ref[...], preferred_element_type=jnp.float32)
                     + bq_ref[...])

    for gi_ in range(G):
        q_b = q_sc[pl.ds(g * GS + gi_ * S, S), :].astype(bf)
        upd_sc[pl.ds(g * GS + gi_ * S, S), :] = _attend(
            q_b, k_ref[gi_], v_ref[gi_])

    @pl.when(g == NG - 1)
    def _global():
        slots = _gru_ff(upd_sc[...], slots_sc[...], *gru_args)

        @pl.when(j < 1)
        def _next():
            slots_sc[...] = slots
            q_sc[...] = (jnp.dot(
                _ln(slots, g_s_ref[...], b_s_ref[...]).astype(bf),
                WqT_ref[...], preferred_element_type=jnp.float32)
                + bq_ref[...])

        @pl.when(j == 1)
        def _emit():
            out_ref[...] = slots.reshape(B, S, D)


@jax.jit
def kernel(x, slots_noise, mu, logsigma, Wq, bq, Wk, bk, Wv, bv,
           W_ih, W_hh, b_ih, b_hh, W1, b1, W2, b2,
           g_in, b_in, g_slots, b_slots, g_ff, b_ff):
    bf = jnp.bfloat16
    row = lambda a: a.reshape(1, -1)
    full = lambda s, n: pl.BlockSpec(s, lambda *_: (0,) * n)
    sigma = jnp.exp(logsigma)
    WqT = Wq.T.astype(bf)

    k, v, upd0 = pl.pallas_call(
        _proj_kernel,
        grid=(NG, 2),
        in_specs=[
            pl.BlockSpec((G, NH, D), lambda g, h: (g, h, 0)),
            pl.BlockSpec((G, S, D), lambda g, h: (g, 0, 0)),
            full((1, 1, D), 3), full((1, 1, D), 3),
            full((D, D), 2), full((1, D), 2),
            full((D, D), 2), full((1, D), 2),
            full((D, D), 2), full((1, D), 2),
            full((1, D), 2), full((1, D), 2),
            full((1, D), 2), full((1, D), 2),
        ],
        out_specs=[
            pl.BlockSpec((G, N, D), lambda g, h: (g, 0, 0)),
            pl.BlockSpec((G, N, D), lambda g, h: (g, 0, 0)),
            pl.BlockSpec((G, S, D), lambda g, h: (g, 0, 0)),
        ],
        out_shape=[
            jax.ShapeDtypeStruct((B, N, D), bf),
            jax.ShapeDtypeStruct((B, N, D), bf),
            jax.ShapeDtypeStruct((B, S, D), jnp.float32),
        ],
    )(x, slots_noise, mu, sigma,
      Wk.T.astype(bf), row(bk), Wv.T.astype(bf), row(bv), WqT, row(bq),
      row(g_in), row(b_in), row(g_slots), row(b_slots))

    out = pl.pallas_call(
        _iter_kernel,
        grid=(ITERS - 1, NG),
        in_specs=[
            pl.BlockSpec((G, N, D), lambda j, g: (g, 0, 0)),
            pl.BlockSpec((G, N, D), lambda j, g: (g, 0, 0)),
            full((B, S, D), 3),
            full((B, S, D), 3),
            full((1, 1, D), 3), full((1, 1, D), 3),
            full((D, D), 2), full((1, D), 2),
            full((D, 3 * D), 2), full((D, 3 * D), 2),
            full((1, 3 * D), 2), full((1, 3 * D), 2),
            full((D, H), 2), full((1, H), 2),
            full((H, D), 2), full((1, D), 2),
            full((1, D), 2), full((1, D), 2),
            full((1, D), 2), full((1, D), 2),
        ],
        out_specs=full((B, S, D), 3),
        out_shape=jax.ShapeDtypeStruct((B, S, D), jnp.float32),
        scratch_shapes=[
            pltpu.VMEM((BS, D), jnp.float32),
            pltpu.VMEM((BS, D), jnp.float32),
            pltpu.VMEM((BS, D), jnp.float32),
        ],
    )(k, v, upd0, slots_noise, mu, sigma,
      WqT, row(bq),
      W_ih.T.astype(bf), W_hh.T.astype(bf), row(b_ih), row(b_hh),
      W1.T.astype(bf), row(b1), W2.T.astype(bf), row(b2),
      row(g_slots), row(b_slots), row(g_ff), row(b_ff))
    return out


# speedup vs baseline: 1.8167x; 1.0333x over previous
import jax
import jax.numpy as jnp
from jax.experimental import pallas as pl
from jax.experimental.pallas import tpu as pltpu

B, N, D = 32, 1024, 768
S = 8
H = 1536
ITERS = 3
EPS = 1e-8
G = 4
NG = B // G
NH = N // 2
GS = G * S
BS = B * S
SCALE = D ** -0.5


def _ln(x, g, b):
    m = jnp.mean(x, axis=-1, keepdims=True)
    v = jnp.mean((x - m) ** 2, axis=-1, keepdims=True)
    return (x - m) * jax.lax.rsqrt(v + 1e-5) * g + b


def _norm_rows(x):
    m = jnp.mean(x, axis=-1, keepdims=True)
    v = jnp.mean((x - m) ** 2, axis=-1, keepdims=True)
    return (x - m) * jax.lax.rsqrt(v + 1e-5)


def _soft_part(q_b, k_h, v_h):
    dots = jax.lax.dot_general(
        q_b, k_h, (((1,), (1,)), ((), ())),
        preferred_element_type=jnp.float32) * SCALE
    e = jnp.exp(dots - jnp.max(dots, axis=0, keepdims=True))
    p = e / jnp.sum(e, axis=0, keepdims=True)
    u = jnp.dot(p.astype(jnp.bfloat16), v_h,
                preferred_element_type=jnp.float32)
    return u, jnp.sum(p, axis=1, keepdims=True)


def _attend(q_b, k0, k1, v0, v1):
    u0, s0 = _soft_part(q_b, k0, v0)
    u1, s1 = _soft_part(q_b, k1, v1)
    return (u0 + u1) / (s0 + s1 + N * EPS)


def _proj_kernel(x_ref, noise_ref, mu_ref, sigma_ref,
                 WkT_ref, bk_ref, WvT_ref, bv_ref, WqT_ref, bq_ref,
                 g_s_ref, b_s_ref,
                 k_ref, v_ref, upd0_ref):
    bf = jnp.bfloat16
    h = pl.program_id(1)
    xh = _norm_rows(x_ref[...].reshape(G * NH, D)).astype(bf)
    kh = (jnp.dot(xh, WkT_ref[...], preferred_element_type=jnp.float32)
          + bk_ref[...]).astype(bf)
    vh = (jnp.dot(xh, WvT_ref[...], preferred_element_type=jnp.float32)
          + bv_ref[...]).astype(bf)
    k_ref[pl.ds(h, 1)] = kh.reshape(1, G, NH, D)
    v_ref[pl.ds(h, 1)] = vh.reshape(1, G, NH, D)

    @pl.when(h == 1)
    def _attn0():
        slots0 = mu_ref[0] + sigma_ref[0] * noise_ref[...].reshape(GS, D)
        q0 = (jnp.dot(_ln(slots0, g_s_ref[...], b_s_ref[...]).astype(bf),
                      WqT_ref[...], preferred_element_type=jnp.float32)
              + bq_ref[...]).astype(bf)
        for gi_ in range(G):
            upd0_ref[gi_] = _attend(q0[gi_ * S:(gi_ + 1) * S, :],
                                    k_ref[0, gi_], k_ref[1, gi_],
                                    v_ref[0, gi_], v_ref[1, gi_])


def _gru_ff(upd, slots_prev, WihT_ref, WhhT_ref, bih_ref, bhh_ref,
            W1T_ref, b1_ref, W2T_ref, b2_ref, g_ff_ref, b_ff_ref):
    bf = jnp.bfloat16
    gi = jnp.dot(upd, WihT_ref[...],
                 preferred_element_type=jnp.float32) + bih_ref[...]
    gh = jnp.dot(slots_prev.astype(bf), WhhT_ref[...],
                 preferred_element_type=jnp.float32) + bhh_ref[...]
    r = jax.nn.sigmoid(gi[:, :D] + gh[:, :D])
    z = jax.nn.sigmoid(gi[:, D:2 * D] + gh[:, D:2 * D])
    n_ = jnp.tanh(gi[:, 2 * D:] + r * gh[:, 2 * D:])
    slots = (1.0 - z) * n_ + z * slots_prev
    ffx = _ln(slots, g_ff_ref[...], b_ff_ref[...]).astype(bf)
    h1 = jax.nn.relu(jnp.dot(ffx, W1T_ref[...],
                             preferred_element_type=jnp.float32)
                     + b1_ref[...]).astype(bf)
    return slots + jnp.dot(h1, W2T_ref[...],
                           preferred_element_type=jnp.float32) + b2_ref[...]


def _iter_kernel(k_ref, v_ref, upd0_ref, noise_ref, mu_ref, sigma_ref,
                 WqT_ref, bq_ref, WihT_ref, WhhT_ref, bih_ref, bhh_ref,
                 W1T_ref, b1_ref, W2T_ref, b2_ref,
                 g_s_ref, b_s_ref, g_ff_ref, b_ff_ref,
                 out_ref, slots_sc, upd_sc, q_sc):
    bf = jnp.bfloat16
    j = pl.program_id(0)
    g = pl.program_id(1)
    gru_args = (WihT_ref, WhhT_ref, bih_ref, bhh_ref,
                W1T_ref, b1_ref, W2T_ref, b2_ref, g_ff_ref, b_ff_ref)

    def _q_of(slots):
        return (jnp.dot(_ln(slots, g_s_ref[...], b_s_ref[...]).astype(bf),
                        WqT_ref[...], preferred_element_type=jnp.float32)
                + bq_ref[...]).astype(bf)

    @pl.when(jnp.logical_and(j == 0, g == 0))
    def _init():
        slots0 = mu_ref[0] + sigma_ref[0] * noise_ref[...].reshape(BS, D)
        slots = _gru_ff(upd0_ref[...].reshape(BS, D).astype(bf),
                        slots0, *gru_args)
        slots_sc[...] = slots
        q_sc[...] = _q_of(slots)

    for gi_ in range(G):
        upd_sc[pl.ds(g * GS + gi_ * S, S), :] = _attend(
            q_sc[pl.ds(g * GS + gi_ * S, S), :],
            k_ref[0, gi_], k_ref[1, gi_],
            v_ref[0, gi_], v_ref[1, gi_]).astype(bf)

    @pl.when(g == NG - 1)
    def _global():
        slots = _gru_ff(upd_sc[...], slots_sc[...], *gru_args)

        @pl.when(j < 1)
        def _next():
            slots_sc[...] = slots
            q_sc[...] = _q_of(slots)

        @pl.when(j == 1)
        def _emit():
            out_ref[...] = slots.reshape(B, S, D)


@jax.jit
def kernel(x, slots_noise, mu, logsigma, Wq, bq, Wk, bk, Wv, bv,
           W_ih, W_hh, b_ih, b_hh, W1, b1, W2, b2,
           g_in, b_in, g_slots, b_slots, g_ff, b_ff):
    bf = jnp.bfloat16
    row = lambda a: a.reshape(1, -1)
    full = lambda s, n: pl.BlockSpec(s, lambda *_: (0,) * n)
    sigma = jnp.exp(logsigma)
    WqT = Wq.T.astype(bf)
    WkT_eff = (g_in[:, None] * Wk.T).astype(bf)
    WvT_eff = (g_in[:, None] * Wv.T).astype(bf)
    bk_eff = row(bk + b_in @ Wk.T)
    bv_eff = row(bv + b_in @ Wv.T)

    k, v, upd0 = pl.pallas_call(
        _proj_kernel,
        grid=(NG, 2),
        in_specs=[
            pl.BlockSpec((G, NH, D), lambda g, h: (g, h, 0)),
            pl.BlockSpec((G, S, D), lambda g, h: (g, 0, 0)),
            full((1, 1, D), 3), full((1, 1, D), 3),
            full((D, D), 2), full((1, D), 2),
            full((D, D), 2), full((1, D), 2),
            full((D, D), 2), full((1, D), 2),
            full((1, D), 2), full((1, D), 2),
        ],
        out_specs=[
            pl.BlockSpec((2, G, NH, D), lambda g, h: (0, g, 0, 0)),
            pl.BlockSpec((2, G, NH, D), lambda g, h: (0, g, 0, 0)),
            pl.BlockSpec((G, S, D), lambda g, h: (g, 0, 0)),
        ],
        out_shape=[
            jax.ShapeDtypeStruct((2, B, NH, D), bf),
            jax.ShapeDtypeStruct((2, B, NH, D), bf),
            jax.ShapeDtypeStruct((B, S, D), jnp.float32),
        ],
    )(x, slots_noise, mu, sigma,
      WkT_eff, bk_eff, WvT_eff, bv_eff, WqT, row(bq),
      row(g_slots), row(b_slots))

    out = pl.pallas_call(
        _iter_kernel,
        grid=(ITERS - 1, NG),
        in_specs=[
            pl.BlockSpec((2, G, NH, D), lambda j, g: (0, g, 0, 0)),
            pl.BlockSpec((2, G, NH, D), lambda j, g: (0, g, 0, 0)),
            full((B, S, D), 3),
            full((B, S, D), 3),
            full((1, 1, D), 3), full((1, 1, D), 3),
            full((D, D), 2), full((1, D), 2),
            full((D, 3 * D), 2), full((D, 3 * D), 2),
            full((1, 3 * D), 2), full((1, 3 * D), 2),
            full((D, H), 2), full((1, H), 2),
            full((H, D), 2), full((1, D), 2),
            full((1, D), 2), full((1, D), 2),
            full((1, D), 2), full((1, D), 2),
        ],
        out_specs=full((B, S, D), 3),
        out_shape=jax.ShapeDtypeStruct((B, S, D), jnp.float32),
        scratch_shapes=[
            pltpu.VMEM((BS, D), jnp.float32),
            pltpu.VMEM((BS, D), bf),
            pltpu.VMEM((BS, D), bf),
        ],
    )(k, v, upd0, slots_noise, mu, sigma,
      WqT, row(bq),
      W_ih.T.astype(bf), W_hh.T.astype(bf), row(b_ih), row(b_hh),
      W1.T.astype(bf), row(b1), W2.T.astype(bf), row(b2),
      row(g_slots), row(b_slots), row(g_ff), row(b_ff))
    return out


# speedup vs baseline: 1.8179x; 1.0006x over previous
import jax
import jax.numpy as jnp
from jax.experimental import pallas as pl
from jax.experimental.pallas import tpu as pltpu

B, N, D = 32, 1024, 768
S = 8
H = 1536
ITERS = 3
EPS = 1e-8
G = 4
NG = B // G
NH = N // 2
GS = G * S
BS = B * S
SCALE = D ** -0.5


def _ln(x, g, b):
    m = jnp.mean(x, axis=-1, keepdims=True)
    v = jnp.mean((x - m) ** 2, axis=-1, keepdims=True)
    return (x - m) * jax.lax.rsqrt(v + 1e-5) * g + b


def _norm_rows(x):
    m = jnp.mean(x, axis=-1, keepdims=True)
    v = jnp.mean((x - m) ** 2, axis=-1, keepdims=True)
    return (x - m) * jax.lax.rsqrt(v + 1e-5)


def _soft_part(q_b, k_h, v_h):
    dots = jax.lax.dot_general(
        q_b, k_h, (((1,), (1,)), ((), ())),
        preferred_element_type=jnp.float32) * SCALE
    e = jnp.exp(dots - jnp.max(dots, axis=0, keepdims=True))
    p = e / jnp.sum(e, axis=0, keepdims=True)
    u = jnp.dot(p.astype(jnp.bfloat16), v_h,
                preferred_element_type=jnp.float32)
    return u, jnp.sum(p, axis=1, keepdims=True)


def _attend(q_b, k0, k1, v0, v1):
    u0, s0 = _soft_part(q_b, k0, v0)
    u1, s1 = _soft_part(q_b, k1, v1)
    return (u0 + u1) / (s0 + s1 + N * EPS)


def _proj_kernel(x_ref, noise_ref, mu_ref, sigma_ref,
                 WkT_ref, bk_ref, WvT_ref, bv_ref, WqT_ref, bq_ref,
                 g_s_ref, b_s_ref,
                 k_ref, v_ref, upd0_ref):
    bf = jnp.bfloat16
    h = pl.program_id(1)
    xh = _norm_rows(x_ref[...].reshape(G * NH, D)).astype(bf)
    kh = (jnp.dot(xh, WkT_ref[...], preferred_element_type=jnp.float32)
          + bk_ref[...]).astype(bf)
    vh = (jnp.dot(xh, WvT_ref[...], preferred_element_type=jnp.float32)
          + bv_ref[...]).astype(bf)
    k_ref[pl.ds(h, 1)] = kh.reshape(1, G, NH, D)
    v_ref[pl.ds(h, 1)] = vh.reshape(1, G, NH, D)

    @pl.when(h == 1)
    def _attn0():
        slots0 = mu_ref[0] + sigma_ref[0] * noise_ref[...].reshape(GS, D)
        q0 = (jnp.dot(_ln(slots0, g_s_ref[...], b_s_ref[...]).astype(bf),
                      WqT_ref[...], preferred_element_type=jnp.float32)
              + bq_ref[...]).astype(bf)
        for gi_ in range(G):
            upd0_ref[gi_] = _attend(q0[gi_ * S:(gi_ + 1) * S, :],
                                    k_ref[0, gi_], k_ref[1, gi_],
                                    v_ref[0, gi_], v_ref[1, gi_])


def _gru_ff(upd, slots_prev, WihT_ref, WhhT_ref, bih_ref, bhh_ref,
            W1T_ref, b1_ref, W2T_ref, b2_ref, g_ff_ref, b_ff_ref):
    bf = jnp.bfloat16
    gi = jnp.dot(upd.astype(bf), WihT_ref[...],
                 preferred_element_type=jnp.float32) + bih_ref[...]
    gh = jnp.dot(slots_prev.astype(bf), WhhT_ref[...],
                 preferred_element_type=jnp.float32) + bhh_ref[...]
    r = jax.nn.sigmoid(gi[:, :D] + gh[:, :D])
    z = jax.nn.sigmoid(gi[:, D:2 * D] + gh[:, D:2 * D])
    n_ = jnp.tanh(gi[:, 2 * D:] + r * gh[:, 2 * D:])
    slots = (1.0 - z) * n_ + z * slots_prev
    ffx = _ln(slots, g_ff_ref[...], b_ff_ref[...]).astype(bf)
    h1 = jax.nn.relu(jnp.dot(ffx, W1T_ref[...],
                             preferred_element_type=jnp.float32)
                     + b1_ref[...]).astype(bf)
    return slots + jnp.dot(h1, W2T_ref[...],
                           preferred_element_type=jnp.float32) + b2_ref[...]


def _iter_kernel(k_ref, v_ref, upd0_ref, noise_ref, mu_ref, sigma_ref,
                 WqT_ref, bq_ref, WihT_ref, WhhT_ref, bih_ref, bhh_ref,
                 W1T_ref, b1_ref, W2T_ref, b2_ref,
                 g_s_ref, b_s_ref, g_ff_ref, b_ff_ref,
                 out_ref, slots_sc, upd_sc, q_sc):
    bf = jnp.bfloat16
    j = pl.program_id(0)
    g = pl.program_id(1)
    gru_args = (WihT_ref, WhhT_ref, bih_ref, bhh_ref,
                W1T_ref, b1_ref, W2T_ref, b2_ref, g_ff_ref, b_ff_ref)

    def _q_of(slots):
        return (jnp.dot(_ln(slots, g_s_ref[...], b_s_ref[...]).astype(bf),
                        WqT_ref[...], preferred_element_type=jnp.float32)
                + bq_ref[...])

    @pl.when(jnp.logical_and(j == 0, g == 0))
    def _init():
        slots0 = mu_ref[0] + sigma_ref[0] * noise_ref[...].reshape(BS, D)
        slots = _gru_ff(upd0_ref[...].reshape(BS, D), slots0, *gru_args)
        slots_sc[...] = slots
        q_sc[...] = _q_of(slots)

    for gi_ in range(G):
        upd_sc[pl.ds(g * GS + gi_ * S, S), :] = _attend(
            q_sc[pl.ds(g * GS + gi_ * S, S), :].astype(bf),
            k_ref[0, gi_], k_ref[1, gi_],
            v_ref[0, gi_], v_ref[1, gi_])

    @pl.when(g == NG - 1)
    def _global():
        slots = _gru_ff(upd_sc[...], slots_sc[...], *gru_args)

        @pl.when(j < 1)
        def _next():
            slots_sc[...] = slots
            q_sc[...] = _q_of(slots)

        @pl.when(j == 1)
        def _emit():
            out_ref[...] = slots.reshape(B, S, D)


@jax.jit
def kernel(x, slots_noise, mu, logsigma, Wq, bq, Wk, bk, Wv, bv,
           W_ih, W_hh, b_ih, b_hh, W1, b1, W2, b2,
           g_in, b_in, g_slots, b_slots, g_ff, b_ff):
    bf = jnp.bfloat16
    row = lambda a: a.reshape(1, -1)
    full = lambda s, n: pl.BlockSpec(s, lambda *_: (0,) * n)
    sigma = jnp.exp(logsigma)
    WqT = Wq.T.astype(bf)
    WkT_eff = (g_in[:, None] * Wk.T).astype(bf)
    WvT_eff = (g_in[:, None] * Wv.T).astype(bf)
    bk_eff = row(bk + b_in @ Wk.T)
    bv_eff = row(bv + b_in @ Wv.T)

    k, v, upd0 = pl.pallas_call(
        _proj_kernel,
        grid=(NG, 2),
        in_specs=[
            pl.BlockSpec((G, NH, D), lambda g, h: (g, h, 0)),
            pl.BlockSpec((G, S, D), lambda g, h: (g, 0, 0)),
            full((1, 1, D), 3), full((1, 1, D), 3),
            full((D, D), 2), full((1, D), 2),
            full((D, D), 2), full((1, D), 2),
            full((D, D), 2), full((1, D), 2),
            full((1, D), 2), full((1, D), 2),
        ],
        out_specs=[
            pl.BlockSpec((2, G, NH, D), lambda g, h: (0, g, 0, 0)),
            pl.BlockSpec((2, G, NH, D), lambda g, h: (0, g, 0, 0)),
            pl.BlockSpec((G, S, D), lambda g, h: (g, 0, 0)),
        ],
        out_shape=[
            jax.ShapeDtypeStruct((2, B, NH, D), bf),
            jax.ShapeDtypeStruct((2, B, NH, D), bf),
            jax.ShapeDtypeStruct((B, S, D), jnp.float32),
        ],
    )(x, slots_noise, mu, sigma,
      WkT_eff, bk_eff, WvT_eff, bv_eff, WqT, row(bq),
      row(g_slots), row(b_slots))

    out = pl.pallas_call(
        _iter_kernel,
        grid=(ITERS - 1, NG),
        in_specs=[
            pl.BlockSpec((2, G, NH, D), lambda j, g: (0, g, 0, 0)),
            pl.BlockSpec((2, G, NH, D), lambda j, g: (0, g, 0, 0)),
            full((B, S, D), 3),
            full((B, S, D), 3),
            full((1, 1, D), 3), full((1, 1, D), 3),
            full((D, D), 2), full((1, D), 2),
            full((D, 3 * D), 2), full((D, 3 * D), 2),
            full((1, 3 * D), 2), full((1, 3 * D), 2),
            full((D, H), 2), full((1, H), 2),
            full((H, D), 2), full((1, D), 2),
            full((1, D), 2), full((1, D), 2),
            full((1, D), 2), full((1, D), 2),
        ],
        out_specs=full((B, S, D), 3),
        out_shape=jax.ShapeDtypeStruct((B, S, D), jnp.float32),
        scratch_shapes=[
            pltpu.VMEM((BS, D), jnp.float32),
            pltpu.VMEM((BS, D), jnp.float32),
            pltpu.VMEM((BS, D), jnp.float32),
        ],
    )(k, v, upd0, slots_noise, mu, sigma,
      WqT, row(bq),
      W_ih.T.astype(bf), W_hh.T.astype(bf), row(b_ih), row(b_hh),
      W1.T.astype(bf), row(b1), W2.T.astype(bf), row(b2),
      row(g_slots), row(b_slots), row(g_ff), row(b_ff))
    return out
